# Initial kernel scaffold; baseline (speedup 1.0000x reference)
import jax
import jax.numpy as jnp
from jax.experimental import pallas as pl


def kernel(edge_costs, edge_counter, t12_costs, t13_costs, t23_costs, tri_corr_12, tri_corr_13, tri_corr_23, W1, b1, W2, b2):
    raise NotImplementedError("write your pallas kernel here")


# speedup vs baseline: 1.0168x; 1.0168x over previous
import jax
import jax.numpy as jnp
from jax.experimental import pallas as pl


def _prep_body(ec_ref, cnt_ref, g_ref, ec0_ref):
    ec = ec_ref[...]
    cnt = cnt_ref[...]
    cntf = jnp.maximum(cnt.astype(jnp.float32), 1.0)
    g_ref[...] = ec / cntf
    ec0_ref[...] = jnp.where(cnt > 0, 0.0, ec)


def _prep(edge_costs, edge_counter):
    E = edge_costs.shape[0]
    BLK = 256000
    grid = E // BLK
    assert grid * BLK == E
    return pl.pallas_call(
        _prep_body,
        grid=(grid,),
        in_specs=[pl.BlockSpec((BLK,), lambda i: (i,)),
                  pl.BlockSpec((BLK,), lambda i: (i,))],
        out_specs=[pl.BlockSpec((BLK,), lambda i: (i,)),
                   pl.BlockSpec((BLK,), lambda i: (i,))],
        out_shape=[jax.ShapeDtypeStruct((E,), jnp.float32),
                   jax.ShapeDtypeStruct((E,), jnp.float32)],
    )(edge_costs, edge_counter)


def kernel(edge_costs, edge_counter, t12_costs, t13_costs, t23_costs,
           tri_corr_12, tri_corr_13, tri_corr_23, W1, b1, W2, b2):
    g, ec0 = _prep(edge_costs, edge_counter)
    t12 = t12_costs + jnp.take(g, tri_corr_12, axis=0)
    t13 = t13_costs + jnp.take(g, tri_corr_13, axis=0)
    t23 = t23_costs + jnp.take(g, tri_corr_23, axis=0)
    tri_features = jnp.stack([t12, t13, t23], axis=1)
    h = jnp.maximum(tri_features @ W1 + b1, 0.0)
    delta = h @ W2 + b2
    edge_updates = jnp.zeros_like(ec0)
    edge_updates = edge_updates.at[tri_corr_12].add(delta[:, 0])
    edge_updates = edge_updates.at[tri_corr_13].add(delta[:, 1])
    edge_updates = edge_updates.at[tri_corr_23].add(delta[:, 2])
    ec = ec0 + edge_updates
    return (ec, t12, t13, t23)


# speedup vs baseline: 16.1130x; 15.8465x over previous
import functools

import jax
import jax.numpy as jnp
from jax import lax
from jax.experimental import pallas as pl
from jax.experimental.pallas import tpu as pltpu
from jax.experimental.pallas import tpu_sc as plsc

_INFO = plsc.get_sparse_core_info()
_NC = _INFO.num_cores
_NS = _INFO.num_subcores
_NW = _NC * _NS


def _prep_body(ec_ref, cnt_ref, g_ref, ec0_ref):
    ec = ec_ref[...]
    cnt = cnt_ref[...]
    cntf = jnp.maximum(cnt.astype(jnp.float32), 1.0)
    g_ref[...] = ec / cntf
    ec0_ref[...] = jnp.where(cnt > 0, 0.0, ec)


def _prep(edge_costs, edge_counter):
    n = edge_costs.shape[0]
    blk = 256000
    grid = n // blk
    return pl.pallas_call(
        _prep_body,
        grid=(grid,),
        in_specs=[pl.BlockSpec((blk,), lambda i: (i,)),
                  pl.BlockSpec((blk,), lambda i: (i,))],
        out_specs=[pl.BlockSpec((blk,), lambda i: (i,)),
                   pl.BlockSpec((blk,), lambda i: (i,))],
        out_shape=[jax.ShapeDtypeStruct((n,), jnp.float32),
                   jax.ShapeDtypeStruct((n,), jnp.float32)],
    )(edge_costs, edge_counter)


def _gather3(g, c12, c13, c23):
    t = c12.shape[0]
    tw = t // _NW
    win = 20000
    nwin = tw // win
    assert nwin * win == tw and tw * _NW == t

    mesh = plsc.VectorSubcoreMesh(core_axis_name="c", subcore_axis_name="s")

    def body(g_hbm, c12_hbm, c13_hbm, c23_hbm, o12, o13, o23,
             idx_v, val_v, sem):
        wid = lax.axis_index("s") * _NC + lax.axis_index("c")
        base = wid * tw

        def step(w, carry):
            off = base + w * win
            for (c_hbm, o_hbm) in ((c12_hbm, o12), (c13_hbm, o13),
                                   (c23_hbm, o23)):
                pltpu.sync_copy(c_hbm.at[pl.ds(off, win)], idx_v)
                pltpu.async_copy(g_hbm.at[idx_v], val_v, sem).wait()
                pltpu.sync_copy(val_v, o_hbm.at[pl.ds(off, win)])
            return carry

        lax.fori_loop(0, nwin, step, 0)

    out_t = [jax.ShapeDtypeStruct((t,), jnp.float32)] * 3
    f = pl.kernel(
        body,
        out_type=out_t,
        mesh=mesh,
        scratch_types=[pltpu.VMEM((win,), jnp.int32),
                       pltpu.VMEM((win,), jnp.float32),
                       pltpu.SemaphoreType.DMA],
    )
    return f(g, c12, c13, c23)


def _mlp_body(w1_ref, b1_ref, w2_ref, b2_ref,
              tc12_ref, tc13_ref, tc23_ref, g12_ref, g13_ref, g23_ref,
              o12, o13, o23, d12, d13, d23):
    t12 = tc12_ref[...] + g12_ref[...]
    t13 = tc13_ref[...] + g13_ref[...]
    t23 = tc23_ref[...] + g23_ref[...]
    o12[...] = t12
    o13[...] = t13
    o23[...] = t23
    d0 = jnp.full_like(t12, b2_ref[0])
    d1 = jnp.full_like(t12, b2_ref[1])
    d2 = jnp.full_like(t12, b2_ref[2])
    for j in range(16):
        hj = jnp.maximum(
            t12 * w1_ref[0, j] + t13 * w1_ref[1, j] + t23 * w1_ref[2, j]
            + b1_ref[j], 0.0)
        d0 = d0 + hj * w2_ref[j, 0]
        d1 = d1 + hj * w2_ref[j, 1]
        d2 = d2 + hj * w2_ref[j, 2]
    d12[...] = d0
    d13[...] = d1
    d23[...] = d2


def _mlp(tc12, tc13, tc23, g12, g13, g23, w1, b1, w2, b2):
    t = tc12.shape[0]
    blk = 128000
    grid = t // blk
    assert grid * blk == t
    smem = pl.BlockSpec(memory_space=pltpu.SMEM)
    dspec = pl.BlockSpec((blk,), lambda i: (i,))
    return pl.pallas_call(
        _mlp_body,
        grid=(grid,),
        in_specs=[smem, smem, smem, smem] + [dspec] * 6,
        out_specs=[dspec] * 6,
        out_shape=[jax.ShapeDtypeStruct((t,), jnp.float32)] * 6,
    )(w1, b1, w2, b2, tc12, tc13, tc23, g12, g13, g23)


def _scatter3(ec0, c12, c13, c23, d12, d13, d23):
    e = ec0.shape[0]
    t = c12.shape[0]
    nchunk = 2 * _NC
    ch = e // nchunk
    chs = ch // _NS
    trash = _NS * 128
    tt = t // _NS
    win = 10000
    nwin = tt // win
    assert nwin * win == tt and ch * nchunk == e and chs * _NS == ch

    mesh = plsc.VectorSubcoreMesh(core_axis_name="c", subcore_axis_name="s")

    def body(ec0_hbm, c12_hbm, c13_hbm, c23_hbm, d12_hbm, d13_hbm, d23_hbm,
             ec_out, acc_sh, idx_v, val_v, oidx_v, sem):
        cid = lax.axis_index("c")
        sid = lax.axis_index("s")
        tbase = sid * tt
        for r in range(nchunk // _NC):
            chunk = r * _NC + cid
            cbase = chunk * ch
            def stage(p, carry):
                pltpu.sync_copy(
                    ec0_hbm.at[pl.ds(cbase + sid * chs + p * win, win)], val_v)
                pltpu.sync_copy(
                    val_v, acc_sh.at[pl.ds(sid * chs + p * win, win)])
                return carry

            lax.fori_loop(0, chs // win, stage, 0)
            plsc.subcore_barrier()

            def step(w, carry):
                off = tbase + w * win
                for (c_hbm, d_hbm) in ((c12_hbm, d12_hbm),
                                       (c13_hbm, d13_hbm),
                                       (c23_hbm, d23_hbm)):
                    pltpu.sync_copy(c_hbm.at[pl.ds(off, win)], idx_v)
                    pltpu.sync_copy(d_hbm.at[pl.ds(off, win)], val_v)

                    def vec(i, c2):
                        idx = idx_v[pl.ds(i * 16, 16)]
                        il = idx - cbase
                        inr = (il >= 0) & (il < ch)
                        tl = (ch + sid * 128 + lax.rem(i, 8) * 16
                              + lax.iota(jnp.int32, 16))
                        oidx_v[pl.ds(i * 16, 16)] = jnp.where(inr, il, tl)
                        return c2

                    lax.fori_loop(0, win // 16, vec, 0)
                    pltpu.sync_copy(val_v, acc_sh.at[oidx_v], add=True)
                return carry

            lax.fori_loop(0, nwin, step, 0)
            plsc.subcore_barrier()
            def unstage(p, carry):
                pltpu.sync_copy(
                    acc_sh.at[pl.ds(sid * chs + p * win, win)], val_v)
                pltpu.sync_copy(
                    val_v, ec_out.at[pl.ds(cbase + sid * chs + p * win, win)])
                return carry

            lax.fori_loop(0, chs // win, unstage, 0)
            plsc.subcore_barrier()

    f = pl.kernel(
        body,
        out_type=jax.ShapeDtypeStruct((e,), jnp.float32),
        mesh=mesh,
        scratch_types=[pltpu.VMEM_SHARED((ch + trash,), jnp.float32),
                       pltpu.VMEM((win,), jnp.int32),
                       pltpu.VMEM((win,), jnp.float32),
                       pltpu.VMEM((win,), jnp.int32),
                       pltpu.SemaphoreType.DMA],
    )
    return f(ec0, c12, c13, c23, d12, d13, d23)


def kernel(edge_costs, edge_counter, t12_costs, t13_costs, t23_costs,
           tri_corr_12, tri_corr_13, tri_corr_23, W1, b1, W2, b2):
    g, ec0 = _prep(edge_costs, edge_counter)
    g12, g13, g23 = _gather3(g, tri_corr_12, tri_corr_13, tri_corr_23)
    t12, t13, t23, d12, d13, d23 = _mlp(
        t12_costs, t13_costs, t23_costs, g12, g13, g23, W1, b1, W2, b2)
    ec = _scatter3(ec0, tri_corr_12, tri_corr_13, tri_corr_23, d12, d13, d23)
    return (ec, t12, t13, t23)
